# baseline (speedup 1.0000x reference)
import functools

import jax
import jax.numpy as jnp
from jax.experimental import pallas as pl
from jax.experimental.pallas import tpu as pltpu

N = 4096
D = 512
BM = 256
BN = 512
NEG = -9.0e15
HIGH = jax.lax.Precision.HIGHEST


def _proj_kernel(x_ref, kw_ref, vw_ref, r4_ref, kh_ref, vh_ref, b_ref):
    x = x_ref[...]
    kh = jnp.dot(x, kw_ref[...])
    kh_ref[...] = kh
    vh_ref[...] = jnp.dot(x, vw_ref[...])
    rota = jnp.dot(kh, r4_ref[...])
    cols = jax.lax.broadcasted_iota(jnp.int32, rota.shape, 1)
    masked = jnp.where(cols < 4, rota, jnp.float32(-3.0e38))
    m = jnp.max(masked, axis=1, keepdims=True)
    idx = jnp.min(jnp.where(masked == m, cols, 128), axis=1, keepdims=True)
    b_ref[...] = idx.astype(jnp.float32)


def _attn_kernel(khr_ref, khc_ref, vhc_ref, adj_ref, br_ref, bc_ref,
                 out_ref, m_ref, l_ref, acc_ref, *, nj):
    j = pl.program_id(1)

    @pl.when(j == 0)
    def _init():
        m_ref[...] = jnp.full_like(m_ref, -jnp.inf)
        l_ref[...] = jnp.zeros_like(l_ref)
        acc_ref[...] = jnp.zeros_like(acc_ref)

    s = jax.lax.dot_general(
        khr_ref[...], khc_ref[...],
        dimension_numbers=(((1,), (1,)), ((), ())),
    ) * jax.lax.rsqrt(jnp.float32(D))
    same = br_ref[...] == bc_ref[...]
    mask = jnp.logical_and(same, adj_ref[...].astype(jnp.int32) > 0)
    s = jnp.where(mask, s, NEG)

    m_prev = m_ref[...]
    m_cur = jnp.max(s, axis=1, keepdims=True)
    m_new = jnp.maximum(m_prev, m_cur)
    alpha = jnp.exp(m_prev - m_new)
    p = jnp.exp(s - m_new)
    l_ref[...] = l_ref[...] * alpha + jnp.sum(p, axis=1, keepdims=True)
    acc_ref[...] = acc_ref[...] * alpha + jnp.dot(p, vhc_ref[...])
    m_ref[...] = m_new

    @pl.when(j == nj - 1)
    def _finalize():
        h = acc_ref[...] / l_ref[...]
        out_ref[...] = jnp.where(h > 0, h, jnp.exp(h) - 1.0)


def kernel(input, adj, kW, vW):
    x = input
    n, d = x.shape
    rot_key = jax.random.key(3933)
    r2 = jax.random.normal(rot_key, (1, d, 2), dtype=jnp.float32)[0]
    r4 = jnp.concatenate([r2, -r2], axis=1)
    r4p = jnp.zeros((d, 128), jnp.float32).at[:, :4].set(r4)
    adj8 = adj.astype(jnp.int8)

    kh, vh, buckets = pl.pallas_call(
        _proj_kernel,
        grid=(n // BM,),
        in_specs=[
            pl.BlockSpec((BM, d), lambda i: (i, 0)),
            pl.BlockSpec((d, d), lambda i: (0, 0)),
            pl.BlockSpec((d, d), lambda i: (0, 0)),
            pl.BlockSpec((d, 128), lambda i: (0, 0)),
        ],
        out_specs=[
            pl.BlockSpec((BM, d), lambda i: (i, 0)),
            pl.BlockSpec((BM, d), lambda i: (i, 0)),
            pl.BlockSpec((BM, 1), lambda i: (i, 0)),
        ],
        out_shape=[
            jax.ShapeDtypeStruct((n, d), jnp.float32),
            jax.ShapeDtypeStruct((n, d), jnp.float32),
            jax.ShapeDtypeStruct((n, 1), jnp.float32),
        ],
    )(x, kW, vW, r4p)

    b_cols = buckets.reshape(1, n)
    nj = n // BN
    out = pl.pallas_call(
        functools.partial(_attn_kernel, nj=nj),
        grid=(n // BM, nj),
        in_specs=[
            pl.BlockSpec((BM, d), lambda i, j: (i, 0)),
            pl.BlockSpec((BN, d), lambda i, j: (j, 0)),
            pl.BlockSpec((BN, d), lambda i, j: (j, 0)),
            pl.BlockSpec((BM, BN), lambda i, j: (i, j)),
            pl.BlockSpec((BM, 1), lambda i, j: (i, 0)),
            pl.BlockSpec((1, BN), lambda i, j: (0, j)),
        ],
        out_specs=pl.BlockSpec((BM, d), lambda i, j: (i, 0)),
        out_shape=jax.ShapeDtypeStruct((n, d), jnp.float32),
        scratch_shapes=[
            pltpu.VMEM((BM, 1), jnp.float32),
            pltpu.VMEM((BM, 1), jnp.float32),
            pltpu.VMEM((BM, d), jnp.float32),
        ],
    )(kh, kh, vh, adj8, buckets, b_cols)
    return out


# speedup vs baseline: 1.0030x; 1.0030x over previous
import functools

import jax
import jax.numpy as jnp
from jax.experimental import pallas as pl
from jax.experimental.pallas import tpu as pltpu

N = 4096
D = 512
BM = 256
BN = 512
NEG = -9.0e15
HIGH = jax.lax.Precision.HIGHEST


def _proj_kernel(x_ref, kw_ref, vw_ref, r4_ref, kh_ref, vh_ref, b_ref):
    x = x_ref[...]
    kh = jnp.dot(x, kw_ref[...])
    kh_ref[...] = kh
    vh_ref[...] = jnp.dot(x, vw_ref[...])
    rota = jnp.dot(kh, r4_ref[...])
    cols = jax.lax.broadcasted_iota(jnp.int32, rota.shape, 1)
    masked = jnp.where(cols < 4, rota, jnp.float32(-3.0e38))
    m = jnp.max(masked, axis=1, keepdims=True)
    idx = jnp.min(jnp.where(masked == m, cols, 128), axis=1, keepdims=True)
    b_ref[...] = idx.astype(jnp.float32)


def _attn_kernel(khr_ref, khc_ref, vhc_ref, adj_ref, br_ref, bc_ref,
                 out_ref, m_ref, l_ref, acc_ref, *, nj):
    j = pl.program_id(1)

    @pl.when(j == 0)
    def _init():
        m_ref[...] = jnp.full_like(m_ref, -jnp.inf)
        l_ref[...] = jnp.zeros_like(l_ref)
        acc_ref[...] = jnp.zeros_like(acc_ref)

    s = jax.lax.dot_general(
        khr_ref[...], khc_ref[...],
        dimension_numbers=(((1,), (1,)), ((), ())),
    ) * jax.lax.rsqrt(jnp.float32(D))
    same = br_ref[...] == bc_ref[...]
    mask = jnp.logical_and(same, adj_ref[...].astype(jnp.int32) > 0)
    s = jnp.where(mask, s, NEG)

    m_prev = m_ref[...]
    m_cur = jnp.max(s, axis=1, keepdims=True)
    m_new = jnp.maximum(m_prev, m_cur)
    alpha = jnp.exp(m_prev - m_new)
    p = jnp.exp(s - m_new)
    l_ref[...] = l_ref[...] * alpha + jnp.sum(p, axis=1, keepdims=True)
    acc_ref[...] = acc_ref[...] * alpha + jnp.dot(p, vhc_ref[...])
    m_ref[...] = m_new

    @pl.when(j == nj - 1)
    def _finalize():
        h = acc_ref[...] / l_ref[...]
        out_ref[...] = jnp.where(h > 0, h, jnp.exp(h) - 1.0)


def kernel(input, adj, kW, vW):
    x = input
    n, d = x.shape
    rot_key = jax.random.key(3933)
    r2 = jax.random.normal(rot_key, (1, d, 2), dtype=jnp.float32)[0]
    r4 = jnp.concatenate([r2, -r2], axis=1)
    r4p = jnp.zeros((d, 128), jnp.float32).at[:, :4].set(r4)
    adj8 = adj.astype(jnp.int8)

    kh, vh, buckets = pl.pallas_call(
        _proj_kernel,
        grid=(n // BM,),
        in_specs=[
            pl.BlockSpec((BM, d), lambda i: (i, 0)),
            pl.BlockSpec((d, d), lambda i: (0, 0)),
            pl.BlockSpec((d, d), lambda i: (0, 0)),
            pl.BlockSpec((d, 128), lambda i: (0, 0)),
        ],
        out_specs=[
            pl.BlockSpec((BM, d), lambda i: (i, 0)),
            pl.BlockSpec((BM, d), lambda i: (i, 0)),
            pl.BlockSpec((BM, 1), lambda i: (i, 0)),
        ],
        out_shape=[
            jax.ShapeDtypeStruct((n, d), jnp.float32),
            jax.ShapeDtypeStruct((n, d), jnp.float32),
            jax.ShapeDtypeStruct((n, 1), jnp.float32),
        ],
        compiler_params=pltpu.CompilerParams(
            dimension_semantics=("parallel",)),
    )(x, kW, vW, r4p)

    b_cols = buckets.reshape(1, n)
    nj = n // BN
    out = pl.pallas_call(
        functools.partial(_attn_kernel, nj=nj),
        grid=(n // BM, nj),
        in_specs=[
            pl.BlockSpec((BM, d), lambda i, j: (i, 0)),
            pl.BlockSpec((BN, d), lambda i, j: (j, 0)),
            pl.BlockSpec((BN, d), lambda i, j: (j, 0)),
            pl.BlockSpec((BM, BN), lambda i, j: (i, j)),
            pl.BlockSpec((BM, 1), lambda i, j: (i, 0)),
            pl.BlockSpec((1, BN), lambda i, j: (0, j)),
        ],
        out_specs=pl.BlockSpec((BM, d), lambda i, j: (i, 0)),
        out_shape=jax.ShapeDtypeStruct((n, d), jnp.float32),
        scratch_shapes=[
            pltpu.VMEM((BM, 1), jnp.float32),
            pltpu.VMEM((BM, 1), jnp.float32),
            pltpu.VMEM((BM, d), jnp.float32),
        ],
        compiler_params=pltpu.CompilerParams(
            dimension_semantics=("parallel", "arbitrary")),
    )(kh, kh, vh, adj8, buckets, b_cols)
    return out


# speedup vs baseline: 1.0933x; 1.0900x over previous
import functools

import jax
import jax.numpy as jnp
from jax.experimental import pallas as pl
from jax.experimental.pallas import tpu as pltpu

N = 4096
D = 512
BM = 256
BN = 512
NEG = -9.0e15
HIGH = jax.lax.Precision.HIGHEST


def _proj_kernel(x_ref, kw_ref, vw_ref, r4_ref, kh_ref, vh_ref, b_ref):
    x = x_ref[...]
    kh = jnp.dot(x, kw_ref[...])
    kh_ref[...] = kh.astype(jnp.bfloat16)
    vh_ref[...] = jnp.dot(x, vw_ref[...]).astype(jnp.bfloat16)
    rota = jnp.dot(kh, r4_ref[...])
    cols = jax.lax.broadcasted_iota(jnp.int32, rota.shape, 1)
    masked = jnp.where(cols < 4, rota, jnp.float32(-3.0e38))
    m = jnp.max(masked, axis=1, keepdims=True)
    idx = jnp.min(jnp.where(masked == m, cols, 128), axis=1, keepdims=True)
    b_ref[...] = idx.astype(jnp.float32)


def _attn_kernel(khr_ref, khc_ref, vhc_ref, adj_ref, br_ref, bc_ref,
                 out_ref, m_ref, l_ref, acc_ref, *, nj):
    j = pl.program_id(1)

    @pl.when(j == 0)
    def _init():
        m_ref[...] = jnp.full_like(m_ref, -jnp.inf)
        l_ref[...] = jnp.zeros_like(l_ref)
        acc_ref[...] = jnp.zeros_like(acc_ref)

    s = jax.lax.dot_general(
        khr_ref[...], khc_ref[...],
        dimension_numbers=(((1,), (1,)), ((), ())),
        preferred_element_type=jnp.float32,
    ) * jax.lax.rsqrt(jnp.float32(D))
    same = br_ref[...] == bc_ref[...]
    mask = jnp.logical_and(same, adj_ref[...].astype(jnp.int32) > 0)
    s = jnp.where(mask, s, NEG)

    m_prev = m_ref[...]
    m_cur = jnp.max(s, axis=1, keepdims=True)
    m_new = jnp.maximum(m_prev, m_cur)
    alpha = jnp.exp(m_prev - m_new)
    p = jnp.exp(s - m_new)
    l_ref[...] = l_ref[...] * alpha + jnp.sum(p, axis=1, keepdims=True)
    acc_ref[...] = acc_ref[...] * alpha + jnp.dot(p.astype(jnp.bfloat16), vhc_ref[...],
                                                  preferred_element_type=jnp.float32)
    m_ref[...] = m_new

    @pl.when(j == nj - 1)
    def _finalize():
        h = acc_ref[...] / l_ref[...]
        out_ref[...] = jnp.where(h > 0, h, jnp.exp(h) - 1.0)


def kernel(input, adj, kW, vW):
    x = input
    n, d = x.shape
    rot_key = jax.random.key(3933)
    r2 = jax.random.normal(rot_key, (1, d, 2), dtype=jnp.float32)[0]
    r4 = jnp.concatenate([r2, -r2], axis=1)
    r4p = jnp.zeros((d, 128), jnp.float32).at[:, :4].set(r4)
    adj8 = adj.astype(jnp.int8)

    kh, vh, buckets = pl.pallas_call(
        _proj_kernel,
        grid=(n // BM,),
        in_specs=[
            pl.BlockSpec((BM, d), lambda i: (i, 0)),
            pl.BlockSpec((d, d), lambda i: (0, 0)),
            pl.BlockSpec((d, d), lambda i: (0, 0)),
            pl.BlockSpec((d, 128), lambda i: (0, 0)),
        ],
        out_specs=[
            pl.BlockSpec((BM, d), lambda i: (i, 0)),
            pl.BlockSpec((BM, d), lambda i: (i, 0)),
            pl.BlockSpec((BM, 1), lambda i: (i, 0)),
        ],
        out_shape=[
            jax.ShapeDtypeStruct((n, d), jnp.bfloat16),
            jax.ShapeDtypeStruct((n, d), jnp.bfloat16),
            jax.ShapeDtypeStruct((n, 1), jnp.float32),
        ],
        compiler_params=pltpu.CompilerParams(
            dimension_semantics=("parallel",)),
    )(x, kW, vW, r4p)

    b_cols = buckets.reshape(1, n)
    nj = n // BN
    out = pl.pallas_call(
        functools.partial(_attn_kernel, nj=nj),
        grid=(n // BM, nj),
        in_specs=[
            pl.BlockSpec((BM, d), lambda i, j: (i, 0)),
            pl.BlockSpec((BN, d), lambda i, j: (j, 0)),
            pl.BlockSpec((BN, d), lambda i, j: (j, 0)),
            pl.BlockSpec((BM, BN), lambda i, j: (i, j)),
            pl.BlockSpec((BM, 1), lambda i, j: (i, 0)),
            pl.BlockSpec((1, BN), lambda i, j: (0, j)),
        ],
        out_specs=pl.BlockSpec((BM, d), lambda i, j: (i, 0)),
        out_shape=jax.ShapeDtypeStruct((n, d), jnp.float32),
        scratch_shapes=[
            pltpu.VMEM((BM, 1), jnp.float32),
            pltpu.VMEM((BM, 1), jnp.float32),
            pltpu.VMEM((BM, d), jnp.float32),
        ],
        compiler_params=pltpu.CompilerParams(
            dimension_semantics=("parallel", "arbitrary")),
    )(kh, kh, vh, adj8, buckets, b_cols)
    return out


# speedup vs baseline: 1.9411x; 1.7756x over previous
import jax
import jax.numpy as jnp
from jax.experimental import pallas as pl
from jax.experimental.pallas import tpu as pltpu

N = 4096
D = 512
BM = 256
BN = 512
NEG = -9.0e15


def _proj_kernel(x_ref, kw_ref, vw_ref, r4_ref, kh_ref, vh_ref, b_ref):
    x = x_ref[...]
    kh = jnp.dot(x, kw_ref[...])
    kh_ref[...] = kh.astype(jnp.bfloat16)
    vh_ref[...] = jnp.dot(x, vw_ref[...]).astype(jnp.bfloat16)
    rota = jnp.dot(kh, r4_ref[...])
    cols = jax.lax.broadcasted_iota(jnp.int32, rota.shape, 1)
    masked = jnp.where(cols < 4, rota, jnp.float32(-3.0e38))
    m = jnp.max(masked, axis=1, keepdims=True)
    idx = jnp.min(jnp.where(masked == m, cols, 128), axis=1, keepdims=True)
    b_ref[...] = idx.astype(jnp.float32)


def _attn_kernel(khr_ref, khc_ref, vhc_ref, adj_ref, br_ref, bc_ref,
                 out_ref, s_ref, p_ref):
    c = jax.lax.rsqrt(jnp.float32(D))
    br = br_ref[...]
    khr = khr_ref[...]
    for j in range(N // BN):
        sl = pl.ds(j * BN, BN)
        s = jax.lax.dot_general(
            khr, khc_ref[sl, :],
            dimension_numbers=(((1,), (1,)), ((), ())),
            preferred_element_type=jnp.float32,
        )
        mask = jnp.logical_and(br == bc_ref[:, sl],
                               adj_ref[:, sl].astype(jnp.int32) > 0)
        s_ref[:, sl] = jnp.where(mask, s * c, NEG)
    s_all = s_ref[...]
    m = jnp.max(s_all, axis=1, keepdims=True)
    e = jnp.exp(s_all - m)
    l = jnp.sum(e, axis=1, keepdims=True)
    p_ref[...] = e.astype(jnp.bfloat16)
    acc = jnp.dot(p_ref[...], vhc_ref[...],
                  preferred_element_type=jnp.float32)
    h = acc / l
    out_ref[...] = jnp.where(h > 0, h, jnp.exp(h) - 1.0)


def kernel(input, adj, kW, vW):
    x = input
    n, d = x.shape
    rot_key = jax.random.key(3933)
    r2 = jax.random.normal(rot_key, (1, d, 2), dtype=jnp.float32)[0]
    r4 = jnp.concatenate([r2, -r2], axis=1)
    r4p = jnp.zeros((d, 128), jnp.float32).at[:, :4].set(r4)
    adj8 = adj.astype(jnp.int8)

    kh, vh, buckets = pl.pallas_call(
        _proj_kernel,
        grid=(n // BM,),
        in_specs=[
            pl.BlockSpec((BM, d), lambda i: (i, 0)),
            pl.BlockSpec((d, d), lambda i: (0, 0)),
            pl.BlockSpec((d, d), lambda i: (0, 0)),
            pl.BlockSpec((d, 128), lambda i: (0, 0)),
        ],
        out_specs=[
            pl.BlockSpec((BM, d), lambda i: (i, 0)),
            pl.BlockSpec((BM, d), lambda i: (i, 0)),
            pl.BlockSpec((BM, 1), lambda i: (i, 0)),
        ],
        out_shape=[
            jax.ShapeDtypeStruct((n, d), jnp.bfloat16),
            jax.ShapeDtypeStruct((n, d), jnp.bfloat16),
            jax.ShapeDtypeStruct((n, 1), jnp.float32),
        ],
        compiler_params=pltpu.CompilerParams(
            dimension_semantics=("parallel",)),
    )(x, kW, vW, r4p)

    b_cols = buckets.reshape(1, n)
    out = pl.pallas_call(
        _attn_kernel,
        grid=(n // BM,),
        in_specs=[
            pl.BlockSpec((BM, d), lambda i: (i, 0)),
            pl.BlockSpec((n, d), lambda i: (0, 0)),
            pl.BlockSpec((n, d), lambda i: (0, 0)),
            pl.BlockSpec((BM, n), lambda i: (i, 0)),
            pl.BlockSpec((BM, 1), lambda i: (i, 0)),
            pl.BlockSpec((1, n), lambda i: (0, 0)),
        ],
        out_specs=pl.BlockSpec((BM, d), lambda i: (i, 0)),
        out_shape=jax.ShapeDtypeStruct((n, d), jnp.float32),
        scratch_shapes=[
            pltpu.VMEM((BM, n), jnp.float32),
            pltpu.VMEM((BM, n), jnp.bfloat16),
        ],
        compiler_params=pltpu.CompilerParams(
            dimension_semantics=("parallel",)),
    )(kh, kh, vh, adj8, buckets, b_cols)
    return out
